# baseline (speedup 1.0000x reference)
import functools

import jax
import jax.numpy as jnp
from jax.experimental import pallas as pl
from jax.experimental.pallas import tpu as pltpu

_NEG = -9e15


def _proj1_body(x_ref, w_ref, a_ref, wh_ref, ff_ref, m_ref):
    wh = jnp.dot(x_ref[...], w_ref[...], preferred_element_type=jnp.float32)
    wh_ref[...] = wh
    ff = jnp.dot(wh, a_ref[...], preferred_element_type=jnp.float32)
    ff_ref[...] = ff
    m_ref[...] = jnp.max(ff[:, 1]).reshape(1, 1)


def _proj2_body(h_ref, wmu_ref, amu_ref, wsig_ref, asig_ref,
                whmu_ref, ffmu_ref, mmu_ref, whsig_ref, ffsig_ref, msig_ref):
    h = h_ref[...]
    whmu = jnp.dot(h, wmu_ref[...], preferred_element_type=jnp.float32)
    whmu_ref[...] = whmu
    ffmu = jnp.dot(whmu, amu_ref[...], preferred_element_type=jnp.float32)
    ffmu_ref[...] = ffmu
    mmu_ref[...] = jnp.max(ffmu[:, 1]).reshape(1, 1)
    whsig = jnp.dot(h, wsig_ref[...], preferred_element_type=jnp.float32)
    whsig_ref[...] = whsig
    ffsig = jnp.dot(whsig, asig_ref[...], preferred_element_type=jnp.float32)
    ffsig_ref[...] = ffsig
    msig_ref[...] = jnp.max(ffsig[:, 1]).reshape(1, 1)


def _leaky(t):
    return jnp.maximum(t, 0.2 * t)


def _att_unnorm(adj, f1, f2, m):
    c = _leaky(f1 + m)
    u = f1 - c
    v = 0.2 * f1 - c
    w = 0.2 * f2
    return jnp.exp(jnp.maximum(u + f2, v + w)) * adj


def _norm(o, d):
    s = o[:, d:d + 1]
    s = jnp.where(s == 0.0, 1.0, s)
    return o[:, :d] / s


def _agg1_body(adj_ref, f1_ref, f2_ref, m_ref, wh_ref, out_ref, pk_ref, *, d):
    adj = adj_ref[...]
    p = _att_unnorm(adj, f1_ref[...], f2_ref[...], m_ref[0, 0])
    o = jnp.dot(p, wh_ref[...], preferred_element_type=jnp.float32)
    out_ref[...] = jnp.maximum(_norm(o, d), 0.0)
    r, n = adj.shape
    m32 = adj.astype(jnp.int32).reshape(r // 8, 8, n)
    k = jax.lax.broadcasted_iota(jnp.int32, (1, 8, 1), 1)
    pk_ref[...] = jnp.sum(m32 << k, axis=1).astype(jnp.uint8).reshape(
        1, r // 8, n)


def _agg23_body(pk_ref, f1mu_ref, f2mu_ref, mmu_ref, whmu_ref,
                f1sig_ref, f2sig_ref, msig_ref, whsig_ref, mu_ref, sig_ref,
                *, d):
    _, rp, n = pk_ref.shape
    pu = pk_ref[...].astype(jnp.int32).reshape(rp, n)
    k = jax.lax.broadcasted_iota(jnp.int32, (1, 8, 1), 1)
    bits = (pu[:, None, :] >> k) & 1
    adj = bits.astype(jnp.float32).reshape(rp * 8, n)
    pmu = _att_unnorm(adj, f1mu_ref[...], f2mu_ref[...], mmu_ref[0, 0])
    omu = jnp.dot(pmu, whmu_ref[...], preferred_element_type=jnp.float32)
    mu_ref[...] = _norm(omu, d)
    psig = _att_unnorm(adj, f1sig_ref[...], f2sig_ref[...], msig_ref[0, 0])
    osig = jnp.dot(psig, whsig_ref[...], preferred_element_type=jnp.float32)
    sig_ref[...] = jnp.exp(_norm(osig, d))


def kernel(x, adj, W1, a1, W_mu, a_mu, W_sig, a_sig):
    n, nf = x.shape
    nh = W1.shape[1]
    nl = W_mu.shape[1]
    f32 = jnp.float32

    a1p = jnp.concatenate([a1[:nh], a1[nh:]], axis=1)
    amup = jnp.concatenate([a_mu[:nl], a_mu[nl:]], axis=1)
    asigp = jnp.concatenate([a_sig[:nl], a_sig[nl:]], axis=1)

    wh1, ff1, m1 = pl.pallas_call(
        _proj1_body,
        out_shape=[jax.ShapeDtypeStruct((n, nh), f32),
                   jax.ShapeDtypeStruct((n, 2), f32),
                   jax.ShapeDtypeStruct((1, 1), f32)],
    )(x, W1, a1p)

    R = 200
    grid = (n // R,)
    row_spec = lambda c: pl.BlockSpec((R, c), lambda i: (i, 0))
    full_spec = lambda r, c: pl.BlockSpec((r, c), lambda i: (0, 0))

    f1_1 = ff1[:, 0:1]
    f2_1 = ff1[:, 1:2].reshape(1, n)

    ones_col = jnp.ones((n, 1), f32)
    wh1e = jnp.concatenate([wh1, ones_col], axis=1)

    h, adjp = pl.pallas_call(
        functools.partial(_agg1_body, d=nh),
        grid=grid,
        in_specs=[row_spec(n), row_spec(1), full_spec(1, n), full_spec(1, 1),
                  full_spec(n, nh + 1)],
        out_specs=[row_spec(nh),
                   pl.BlockSpec((1, R // 8, n), lambda i: (i, 0, 0))],
        out_shape=[jax.ShapeDtypeStruct((n, nh), f32),
                   jax.ShapeDtypeStruct((n // R, R // 8, n), jnp.uint8)],
        compiler_params=pltpu.CompilerParams(
            dimension_semantics=("arbitrary",)),
    )(adj, f1_1, f2_1, m1, wh1e)

    whmu, ffmu, mmu, whsig, ffsig, msig = pl.pallas_call(
        _proj2_body,
        out_shape=[jax.ShapeDtypeStruct((n, nl), f32),
                   jax.ShapeDtypeStruct((n, 2), f32),
                   jax.ShapeDtypeStruct((1, 1), f32),
                   jax.ShapeDtypeStruct((n, nl), f32),
                   jax.ShapeDtypeStruct((n, 2), f32),
                   jax.ShapeDtypeStruct((1, 1), f32)],
    )(h, W_mu, amup, W_sig, asigp)

    whmue = jnp.concatenate([whmu, ones_col], axis=1)
    whsige = jnp.concatenate([whsig, ones_col], axis=1)

    mu, sig = pl.pallas_call(
        functools.partial(_agg23_body, d=nl),
        grid=grid,
        in_specs=[pl.BlockSpec((1, R // 8, n), lambda i: (i, 0, 0)),
                  row_spec(1), full_spec(1, n), full_spec(1, 1),
                  full_spec(n, nl + 1),
                  row_spec(1), full_spec(1, n), full_spec(1, 1),
                  full_spec(n, nl + 1)],
        out_specs=[row_spec(nl), row_spec(nl)],
        out_shape=[jax.ShapeDtypeStruct((n, nl), f32),
                   jax.ShapeDtypeStruct((n, nl), f32)],
        compiler_params=pltpu.CompilerParams(
            dimension_semantics=("arbitrary",)),
    )(adjp, ffmu[:, 0:1], ffmu[:, 1:2].reshape(1, n), mmu, whmue,
      ffsig[:, 0:1], ffsig[:, 1:2].reshape(1, n), msig, whsige)

    return (mu, sig)


# speedup vs baseline: 1.5685x; 1.5685x over previous
import functools

import jax
import jax.numpy as jnp
from jax.experimental import pallas as pl
from jax.experimental.pallas import tpu as pltpu

_NEG = -9e15


def _proj1_body(x_ref, w_ref, a_ref, wh_ref, ff_ref, m_ref):
    wh = jnp.dot(x_ref[...], w_ref[...], preferred_element_type=jnp.float32)
    wh_ref[...] = wh
    ff = jnp.dot(wh, a_ref[...], preferred_element_type=jnp.float32)
    ff_ref[...] = ff
    m_ref[...] = jnp.max(ff[:, 1]).reshape(1, 1)


def _proj2_body(h_ref, wmu_ref, amu_ref, wsig_ref, asig_ref,
                whmu_ref, ffmu_ref, mmu_ref, whsig_ref, ffsig_ref, msig_ref):
    h = h_ref[...]
    whmu = jnp.dot(h, wmu_ref[...], preferred_element_type=jnp.float32)
    whmu_ref[...] = whmu
    ffmu = jnp.dot(whmu, amu_ref[...], preferred_element_type=jnp.float32)
    ffmu_ref[...] = ffmu
    mmu_ref[...] = jnp.max(ffmu[:, 1]).reshape(1, 1)
    whsig = jnp.dot(h, wsig_ref[...], preferred_element_type=jnp.float32)
    whsig_ref[...] = whsig
    ffsig = jnp.dot(whsig, asig_ref[...], preferred_element_type=jnp.float32)
    ffsig_ref[...] = ffsig
    msig_ref[...] = jnp.max(ffsig[:, 1]).reshape(1, 1)


def _leaky(t):
    return jnp.maximum(t, 0.2 * t)


def _att_unnorm(adj, f1, f2, m):
    c = _leaky(f1 + m)
    u = f1 - c
    v = 0.2 * f1 - c
    w = 0.2 * f2
    return jnp.exp(jnp.maximum(u + f2, v + w)) * adj


def _norm(o, d):
    s = o[:, d:d + 1]
    s = jnp.where(s == 0.0, 1.0, s)
    return o[:, :d] / s


def _agg1_body(adj_ref, f1_ref, f2_ref, m_ref, wh_ref, out_ref, *, d):
    p = _att_unnorm(adj_ref[...], f1_ref[...], f2_ref[...], m_ref[0, 0])
    o = jnp.dot(p, wh_ref[...], preferred_element_type=jnp.float32)
    out_ref[...] = jnp.maximum(_norm(o, d), 0.0)


def _agg23_body(adj_ref, f1mu_ref, f2mu_ref, mmu_ref, whmu_ref,
                f1sig_ref, f2sig_ref, msig_ref, whsig_ref, mu_ref, sig_ref,
                *, d):
    adj = adj_ref[...]
    pmu = _att_unnorm(adj, f1mu_ref[...], f2mu_ref[...], mmu_ref[0, 0])
    omu = jnp.dot(pmu, whmu_ref[...], preferred_element_type=jnp.float32)
    mu_ref[...] = _norm(omu, d)
    psig = _att_unnorm(adj, f1sig_ref[...], f2sig_ref[...], msig_ref[0, 0])
    osig = jnp.dot(psig, whsig_ref[...], preferred_element_type=jnp.float32)
    sig_ref[...] = jnp.exp(_norm(osig, d))


def kernel(x, adj, W1, a1, W_mu, a_mu, W_sig, a_sig):
    n, nf = x.shape
    nh = W1.shape[1]
    nl = W_mu.shape[1]
    f32 = jnp.float32

    a1p = jnp.concatenate([a1[:nh], a1[nh:]], axis=1)
    amup = jnp.concatenate([a_mu[:nl], a_mu[nl:]], axis=1)
    asigp = jnp.concatenate([a_sig[:nl], a_sig[nl:]], axis=1)

    wh1, ff1, m1 = pl.pallas_call(
        _proj1_body,
        out_shape=[jax.ShapeDtypeStruct((n, nh), f32),
                   jax.ShapeDtypeStruct((n, 2), f32),
                   jax.ShapeDtypeStruct((1, 1), f32)],
    )(x, W1, a1p)

    R = 400
    grid = (n // R,)
    row_spec = lambda c: pl.BlockSpec((R, c), lambda i: (i, 0))
    full_spec = lambda r, c: pl.BlockSpec((r, c), lambda i: (0, 0))

    f1_1 = ff1[:, 0:1]
    f2_1 = ff1[:, 1:2].reshape(1, n)

    ones_col = jnp.ones((n, 1), f32)
    wh1e = jnp.concatenate([wh1, ones_col], axis=1)

    h = pl.pallas_call(
        functools.partial(_agg1_body, d=nh),
        grid=grid,
        in_specs=[row_spec(n), row_spec(1), full_spec(1, n), full_spec(1, 1),
                  full_spec(n, nh + 1)],
        out_specs=row_spec(nh),
        out_shape=jax.ShapeDtypeStruct((n, nh), f32),
        compiler_params=pltpu.CompilerParams(
            dimension_semantics=("arbitrary",)),
    )(adj, f1_1, f2_1, m1, wh1e)

    whmu, ffmu, mmu, whsig, ffsig, msig = pl.pallas_call(
        _proj2_body,
        out_shape=[jax.ShapeDtypeStruct((n, nl), f32),
                   jax.ShapeDtypeStruct((n, 2), f32),
                   jax.ShapeDtypeStruct((1, 1), f32),
                   jax.ShapeDtypeStruct((n, nl), f32),
                   jax.ShapeDtypeStruct((n, 2), f32),
                   jax.ShapeDtypeStruct((1, 1), f32)],
    )(h, W_mu, amup, W_sig, asigp)

    whmue = jnp.concatenate([whmu, ones_col], axis=1)
    whsige = jnp.concatenate([whsig, ones_col], axis=1)

    mu, sig = pl.pallas_call(
        functools.partial(_agg23_body, d=nl),
        grid=grid,
        in_specs=[row_spec(n),
                  row_spec(1), full_spec(1, n), full_spec(1, 1),
                  full_spec(n, nl + 1),
                  row_spec(1), full_spec(1, n), full_spec(1, 1),
                  full_spec(n, nl + 1)],
        out_specs=[row_spec(nl), row_spec(nl)],
        out_shape=[jax.ShapeDtypeStruct((n, nl), f32),
                   jax.ShapeDtypeStruct((n, nl), f32)],
        compiler_params=pltpu.CompilerParams(
            dimension_semantics=("arbitrary",)),
    )(adj, ffmu[:, 0:1], ffmu[:, 1:2].reshape(1, n), mmu, whmue,
      ffsig[:, 0:1], ffsig[:, 1:2].reshape(1, n), msig, whsige)

    return (mu, sig)
